# baseline (speedup 1.0000x reference)
import functools

import jax
import jax.numpy as jnp
from jax.experimental import pallas as pl
from jax.experimental.pallas import tpu as pltpu
from jax.experimental.pallas import tpu_sc as plsc

B = 32
V = 1_000_000
CHUNK = 400
NCHUNK = V // CHUNK
NCAND = 64
TOPK = 50
INV_TEMP = 1.25
KEY_HI = 0
KEY_LO = 42


def _chunkmax_kernel(x_ref, out_ref):
    out_ref[0, 0, :] = jnp.max(x_ref[0], axis=1)


def _phase_a(x3):
    out = pl.pallas_call(
        _chunkmax_kernel,
        grid=(B,),
        in_specs=[pl.BlockSpec((1, NCHUNK, CHUNK), lambda r: (r, 0, 0))],
        out_specs=pl.BlockSpec((1, 1, NCHUNK), lambda r: (r, 0, 0)),
        out_shape=jax.ShapeDtypeStruct((B, 1, NCHUNK), jnp.float32),
    )(x3)
    return out.reshape(B, NCHUNK)


def _select_kernel(cm_ref, idx_ref):
    v = cm_ref[...]
    lanes = jax.lax.broadcasted_iota(jnp.int32, (B, NCHUNK), 1)
    roff = jax.lax.broadcasted_iota(jnp.int32, (B, 1), 0) * NCHUNK
    for i in range(NCAND):
        m = jnp.argmax(v, axis=1).astype(jnp.int32)
        idx_ref[:, i : i + 1] = m[:, None] + roff
        v = jnp.where(lanes == m[:, None], -jnp.inf, v)


def _phase_b(cm):
    return pl.pallas_call(
        _select_kernel,
        grid=(1,),
        in_specs=[pl.BlockSpec((B, NCHUNK), lambda i: (0, 0))],
        out_specs=pl.BlockSpec((B, NCAND), lambda i: (0, 0)),
        out_shape=jax.ShapeDtypeStruct((B, NCAND), jnp.int32),
    )(cm)


def _rotl(x, d):
    return (x << jnp.uint32(d)) | (x >> jnp.uint32(32 - d))


def _gumbel_bits(flat_u32):
    k0, k1 = jnp.uint32(KEY_HI), jnp.uint32(KEY_LO)
    ks2 = jnp.uint32(KEY_HI ^ KEY_LO ^ 0x1BD11BDA)
    ks = (k0, k1, ks2)
    x0 = jnp.zeros_like(flat_u32) + ks[0]
    x1 = flat_u32 + ks[1]
    rot = ((13, 15, 26, 6), (17, 29, 16, 24))
    for g in range(5):
        for r in rot[g % 2]:
            x0 = x0 + x1
            x1 = _rotl(x1, r)
            x1 = x1 ^ x0
        x0 = x0 + ks[(g + 1) % 3]
        x1 = x1 + ks[(g + 2) % 3] + jnp.uint32(g + 1)
    bits = x0 ^ x1
    fb = pltpu.bitcast((bits >> jnp.uint32(9)) | jnp.uint32(0x3F800000),
                       jnp.float32)
    tiny = jnp.float32(jnp.finfo(jnp.float32).tiny)
    u01 = fb - jnp.float32(1.0)
    u = u01 * (jnp.float32(1.0) - tiny) + tiny
    u = jnp.maximum(tiny, u)
    return -jnp.log(-jnp.log(u))


_NW = 32
_PER_W = B * NCAND // _NW


def _sc_gather_kernel(table_hbm, idx_hbm, out_hbm, idx_v, rows_v, sem):
    wid = jax.lax.axis_index("s") * 2 + jax.lax.axis_index("c")
    base = wid * _PER_W
    pltpu.sync_copy(idx_hbm.at[pl.ds(base, _PER_W)], idx_v)
    pltpu.async_copy(table_hbm.at[idx_v], rows_v, sem).wait()
    pltpu.sync_copy(rows_v, out_hbm.at[pl.ds(base, _PER_W)])


def _phase_c(x3, idx):
    table = x3.reshape(B * NCHUNK, CHUNK)
    mesh = plsc.VectorSubcoreMesh(core_axis_name="c", subcore_axis_name="s")
    k = functools.partial(
        pl.kernel,
        mesh=mesh,
        compiler_params=pltpu.CompilerParams(use_tc_tiling_on_sc=False),
        out_type=jax.ShapeDtypeStruct((B * NCAND, CHUNK), jnp.float32),
        scratch_types=[
            pltpu.VMEM((_PER_W,), jnp.int32),
            pltpu.VMEM((_PER_W, CHUNK), jnp.float32),
            pltpu.SemaphoreType.DMA,
        ],
    )(_sc_gather_kernel)
    return k(table, idx.reshape(-1)).reshape(B, NCAND, CHUNK)


def _finish_kernel(g_ref, idx_ref, out_ref):
    v = g_ref[...]
    flat_i = (idx_ref[...][:, :, None] * CHUNK
              + jax.lax.broadcasted_iota(jnp.int32, (B, NCAND, CHUNK), 2))
    rows = jax.lax.broadcasted_iota(jnp.int32, (B, NCAND, CHUNK), 0)
    cols = flat_i - rows * V
    gum = _gumbel_bits(pltpu.bitcast(flat_i, jnp.uint32))
    score = v * jnp.float32(INV_TEMP) + gum
    w = v
    removed = jnp.zeros((B, 1, 1), jnp.int32)
    t = jnp.full((B, 1, 1), jnp.inf, jnp.float32)
    for _ in range(TOPK):
        m = jnp.max(jnp.max(w, axis=2), axis=1)[:, None, None]
        hit = w == m
        cnt = jnp.sum(jnp.sum(hit.astype(jnp.int32), axis=2), axis=1)
        t = jnp.where(removed < TOPK, m, t)
        w = jnp.where(hit, -jnp.inf, w)
        removed = removed + cnt[:, None, None]
    s = jnp.where(v >= t, score, -jnp.inf)
    best = jnp.max(jnp.max(s, axis=2), axis=1)[:, None, None]
    sample = jnp.max(jnp.max(jnp.where(s == best, cols, -1), axis=2), axis=1)
    out_ref[...] = jnp.broadcast_to(sample[:, None, None], (B, 1, 128))


def _phase_d(g, idx):
    out = pl.pallas_call(
        _finish_kernel,
        grid=(1,),
        in_specs=[
            pl.BlockSpec((B, NCAND, CHUNK), lambda i: (0, 0, 0)),
            pl.BlockSpec((B, NCAND), lambda i: (0, 0)),
        ],
        out_specs=pl.BlockSpec((B, 1, 128), lambda i: (0, 0, 0)),
        out_shape=jax.ShapeDtypeStruct((B, 1, 128), jnp.int32),
    )(g, idx)
    return out[:, 0, :1]


@jax.jit
def kernel(x):
    x3 = x.reshape(B, NCHUNK, CHUNK)
    cm = _phase_a(x3)
    idx = _phase_b(cm)
    g = _phase_c(x3, idx)
    return _phase_d(g, idx)


# speedup vs baseline: 4.9215x; 4.9215x over previous
import functools

import jax
import jax.numpy as jnp
from jax.experimental import pallas as pl
from jax.experimental.pallas import tpu as pltpu
from jax.experimental.pallas import tpu_sc as plsc

B = 32
V = 1_000_000
CHUNK = 512
NCHUNK = V // CHUNK
TAIL = V - NCHUNK * CHUNK
NCAND = 64
NC1 = NCAND + 1
TOPK = 50
INV_TEMP = 1.25
KEY_HI = 0
KEY_LO = 42


def _chunkmax_kernel(x_ref, out_ref):
    out_ref[0, 0, :] = jnp.max(x_ref[0], axis=1)


def _phase_a(x3):
    out = pl.pallas_call(
        _chunkmax_kernel,
        grid=(B,),
        in_specs=[pl.BlockSpec((1, NCHUNK, CHUNK), lambda r: (r, 0, 0))],
        out_specs=pl.BlockSpec((1, 1, NCHUNK), lambda r: (r, 0, 0)),
        out_shape=jax.ShapeDtypeStruct((B, 1, NCHUNK), jnp.float32),
    )(x3)
    return out.reshape(B, NCHUNK)


def _select_kernel(cm_ref, idx_ref):
    v = cm_ref[...]
    lanes = jax.lax.broadcasted_iota(jnp.int32, (B, NCHUNK), 1)
    roff = jax.lax.broadcasted_iota(jnp.int32, (B, 1), 0) * NCHUNK
    for i in range(NCAND):
        m = jnp.argmax(v, axis=1).astype(jnp.int32)
        idx_ref[:, i : i + 1] = m[:, None] + roff
        v = jnp.where(lanes == m[:, None], -jnp.inf, v)
    idx_ref[:, NCAND : NCAND + 1] = roff + NCHUNK


def _phase_b(cm):
    return pl.pallas_call(
        _select_kernel,
        grid=(1,),
        in_specs=[pl.BlockSpec((B, NCHUNK), lambda i: (0, 0))],
        out_specs=pl.BlockSpec((B, NC1), lambda i: (0, 0)),
        out_shape=jax.ShapeDtypeStruct((B, NC1), jnp.int32),
    )(cm)


def _rotl(x, d):
    return (x << jnp.uint32(d)) | (x >> jnp.uint32(32 - d))


def _gumbel_bits(flat_u32):
    k0, k1 = jnp.uint32(KEY_HI), jnp.uint32(KEY_LO)
    ks2 = jnp.uint32(KEY_HI ^ KEY_LO ^ 0x1BD11BDA)
    ks = (k0, k1, ks2)
    x0 = jnp.zeros_like(flat_u32) + ks[0]
    x1 = flat_u32 + ks[1]
    rot = ((13, 15, 26, 6), (17, 29, 16, 24))
    for g in range(5):
        for r in rot[g % 2]:
            x0 = x0 + x1
            x1 = _rotl(x1, r)
            x1 = x1 ^ x0
        x0 = x0 + ks[(g + 1) % 3]
        x1 = x1 + ks[(g + 2) % 3] + jnp.uint32(g + 1)
    bits = x0 ^ x1
    fb = pltpu.bitcast((bits >> jnp.uint32(9)) | jnp.uint32(0x3F800000),
                       jnp.float32)
    tiny = jnp.float32(jnp.finfo(jnp.float32).tiny)
    u01 = fb - jnp.float32(1.0)
    u = u01 * (jnp.float32(1.0) - tiny) + tiny
    u = jnp.maximum(tiny, u)
    return -jnp.log(-jnp.log(u))


_NW = 32
_PER_W = B * NCAND // _NW


def _sc_gather_kernel(table_hbm, idx_hbm, out_hbm, idx_v, rows_v, sem):
    wid = jax.lax.axis_index("s") * 2 + jax.lax.axis_index("c")
    base = wid * _PER_W
    pltpu.sync_copy(idx_hbm.at[pl.ds(base, _PER_W)], idx_v)
    pltpu.async_copy(table_hbm.at[idx_v], rows_v, sem).wait()
    pltpu.sync_copy(rows_v, out_hbm.at[pl.ds(base, _PER_W)])


def _phase_c(table, idx64):
    mesh = plsc.VectorSubcoreMesh(core_axis_name="c", subcore_axis_name="s")
    k = functools.partial(
        pl.kernel,
        mesh=mesh,
        out_type=jax.ShapeDtypeStruct((B * NCAND, CHUNK), jnp.float32),
        scratch_types=[
            pltpu.VMEM((_PER_W,), jnp.int32),
            pltpu.VMEM((_PER_W, CHUNK), jnp.float32),
            pltpu.SemaphoreType.DMA,
        ],
    )(_sc_gather_kernel)
    return k(table, idx64).reshape(B, NCAND, CHUNK)


def _finish_kernel(g_ref, idx_ref, out_ref):
    v = g_ref[...]
    rows = jax.lax.broadcasted_iota(jnp.int32, (B, NC1, CHUNK), 0)
    flat_i = (idx_ref[...][:, :, None] * CHUNK + rows * TAIL
              + jax.lax.broadcasted_iota(jnp.int32, (B, NC1, CHUNK), 2))
    cols = flat_i - rows * V
    gum = _gumbel_bits(pltpu.bitcast(flat_i, jnp.uint32))
    score = v * jnp.float32(INV_TEMP) + gum
    w = v
    removed = jnp.zeros((B, 1, 1), jnp.int32)
    t = jnp.full((B, 1, 1), jnp.inf, jnp.float32)
    for _ in range(TOPK):
        m = jnp.max(jnp.max(w, axis=2), axis=1)[:, None, None]
        hit = w == m
        cnt = jnp.sum(jnp.sum(hit.astype(jnp.int32), axis=2), axis=1)
        t = jnp.where(removed < TOPK, m, t)
        w = jnp.where(hit, -jnp.inf, w)
        removed = removed + cnt[:, None, None]
    s = jnp.where(v >= t, score, -jnp.inf)
    best = jnp.max(jnp.max(s, axis=2), axis=1)[:, None, None]
    sample = jnp.max(jnp.max(jnp.where(s == best, cols, -1), axis=2), axis=1)
    out_ref[...] = jnp.broadcast_to(sample[:, None, None], (B, 1, 128))


def _phase_d(g, idx):
    out = pl.pallas_call(
        _finish_kernel,
        grid=(1,),
        in_specs=[
            pl.BlockSpec((B, NC1, CHUNK), lambda i: (0, 0, 0)),
            pl.BlockSpec((B, NC1), lambda i: (0, 0)),
        ],
        out_specs=pl.BlockSpec((B, 1, 128), lambda i: (0, 0, 0)),
        out_shape=jax.ShapeDtypeStruct((B, 1, 128), jnp.int32),
    )(g, idx)
    return out[:, 0, :1]


@jax.jit
def kernel(x):
    body = x[:, : NCHUNK * CHUNK].reshape(B, NCHUNK, CHUNK)
    tail = x[:, NCHUNK * CHUNK :]
    cm = _phase_a(body)
    idx = _phase_b(cm)
    g = _phase_c(body.reshape(B * NCHUNK, CHUNK), idx[:, :NCAND].reshape(-1))
    tailp = jnp.pad(tail, ((0, 0), (0, CHUNK - TAIL)),
                    constant_values=-jnp.inf)[:, None, :]
    g1 = jnp.concatenate([g, tailp], axis=1)
    return _phase_d(g1, idx)
